# baseline (speedup 1.0000x reference)
import functools

import jax
import jax.numpy as jnp
from jax import lax
from jax.experimental import pallas as pl
from jax.experimental.pallas import tpu as pltpu
from jax.experimental.pallas import tpu_sc as plsc

GRID = 192
G2 = GRID * GRID
G3 = GRID * GRID * GRID
DX = 1.0 / (GRID - 1)
E_IN = 80.0
E_OUT = 2.0
WEIGHT = 10.0
BATCH = 4

NW = 32
CHUNK = 128
L = 16
ROWW = 16
NROWS = G3 // ROWW
RSH = 4
RMASK = ROWW - 1


def _sc_body(n_points, p_per_w, sin_hbm, sout_hbm, xi_hbm, yi_hbm, zi_hbm,
             nx_hbm, ny_hbm, nz_hbm, out_hbm,
             xv, yv, zv, nxv, nyv, nzv,
             rc, rxi, ryi, rzi, rxo, ryo, rzo,
             lc, lxi, lyi, lzi, lxo, lyo, lzo,
             gci, gco, gxi, gyi, gzi, gxo, gyo, gzo,
             accv, sem):
    num_cores = 2
    wid = lax.axis_index("s") * num_cores + lax.axis_index("c")
    base = wid * p_per_w

    pltpu.sync_copy(xi_hbm.at[pl.ds(base, p_per_w)], xv)
    pltpu.sync_copy(yi_hbm.at[pl.ds(base, p_per_w)], yv)
    pltpu.sync_copy(zi_hbm.at[pl.ds(base, p_per_w)], zv)
    pltpu.sync_copy(nx_hbm.at[pl.ds(base, p_per_w)], nxv)
    pltpu.sync_copy(ny_hbm.at[pl.ds(base, p_per_w)], nyv)
    pltpu.sync_copy(nz_hbm.at[pl.ds(base, p_per_w)], nzv)

    n_vec = p_per_w // L

    def idx_body(i, _):
        s = pl.ds(i * L, L)
        x = xv[s]
        y = yv[s]
        z = zv[s]
        f = (x * GRID + y) * GRID + z
        dx = jnp.where(nxv[s] > 0.0, -G2, G2)
        dy = jnp.where(nyv[s] > 0.0, -GRID, GRID)
        dz = jnp.where(nzv[s] > 0.0, -1, 1)
        fxi = f + dx
        fxo = f - dx
        fyi = f + dy
        fyo = f - dy
        fzi = f + dz
        fzo = f - dz
        rc[s] = jnp.right_shift(f, RSH)
        lc[s] = jnp.bitwise_and(f, RMASK)
        rxi[s] = jnp.right_shift(fxi, RSH)
        lxi[s] = jnp.bitwise_and(fxi, RMASK)
        rxo[s] = jnp.right_shift(fxo, RSH)
        lxo[s] = jnp.bitwise_and(fxo, RMASK)
        ryi[s] = jnp.right_shift(fyi, RSH)
        lyi[s] = jnp.bitwise_and(fyi, RMASK)
        ryo[s] = jnp.right_shift(fyo, RSH)
        lyo[s] = jnp.bitwise_and(fyo, RMASK)
        rzi[s] = jnp.right_shift(fzi, RSH)
        lzi[s] = jnp.bitwise_and(fzi, RMASK)
        rzo[s] = jnp.right_shift(fzo, RSH)
        lzo[s] = jnp.bitwise_and(fzo, RMASK)
        return 0

    lax.fori_loop(0, n_vec, idx_body, 0, unroll=2)

    inv_dx = jnp.float32(1.0 / DX)
    e_in = jnp.float32(E_IN)
    e_out = jnp.float32(E_OUT)
    n_chunks = p_per_w // CHUNK
    vpc = CHUNK // L

    acc = jnp.zeros((L,), jnp.float32)
    for b in range(BATCH):
        sin_b = sin_hbm.at[b]
        sout_b = sout_hbm.at[b]

        def chunk_body(r, a, sin_b=sin_b, sout_b=sout_b):
            cs = pl.ds(r * CHUNK, CHUNK)
            copies = (
                pltpu.async_copy(sin_b.at[rc.at[cs]], gci, sem),
                pltpu.async_copy(sout_b.at[rc.at[cs]], gco, sem),
                pltpu.async_copy(sin_b.at[rxi.at[cs]], gxi, sem),
                pltpu.async_copy(sin_b.at[ryi.at[cs]], gyi, sem),
                pltpu.async_copy(sin_b.at[rzi.at[cs]], gzi, sem),
                pltpu.async_copy(sout_b.at[rxo.at[cs]], gxo, sem),
                pltpu.async_copy(sout_b.at[ryo.at[cs]], gyo, sem),
                pltpu.async_copy(sout_b.at[rzo.at[cs]], gzo, sem),
            )
            for c in copies:
                c.wait()

            for i in range(vpc):
                s = pl.ds(r * CHUNK + i * L, L)
                rows = lax.iota(jnp.int32, L) + i * L
                gid = base + r * CHUNK + i * L + lax.iota(jnp.int32, L)
                m = jnp.where(gid < n_points, 1.0, 0.0).astype(jnp.float32)
                cin = plsc.load_gather(gci, [rows, lc[s]])
                cout = plsc.load_gather(gco, [rows, lc[s]])
                uxi = plsc.load_gather(gxi, [rows, lxi[s]])
                uyi = plsc.load_gather(gyi, [rows, lyi[s]])
                uzi = plsc.load_gather(gzi, [rows, lzi[s]])
                vxo = plsc.load_gather(gxo, [rows, lxo[s]])
                vyo = plsc.load_gather(gyo, [rows, lyo[s]])
                vzo = plsc.load_gather(gzo, [rows, lzo[s]])
                wx = jnp.abs(nxv[s]) * inv_dx
                wy = jnp.abs(nyv[s]) * inv_dx
                wz = jnp.abs(nzv[s]) * inv_dx
                t1 = cin - cout
                t2 = (wx * (e_in * (cin - uxi) + e_out * (cout - vxo))
                      + wy * (e_in * (cin - uyi) + e_out * (cout - vyo))
                      + wz * (e_in * (cin - uzi) + e_out * (cout - vzo)))
                a = a + m * (t1 * t1 + t2 * t2)
            return a

        acc = lax.fori_loop(0, n_chunks, chunk_body, acc)

    accv[...] = acc
    pltpu.sync_copy(accv, out_hbm.at[wid])


def kernel(subdomain_in, subdomain_out, normal_x, normal_y, normal_z,
           x_idx, y_idx, z_idx):
    n = x_idx.shape[0]
    p_per_w = ((n + NW * CHUNK - 1) // (NW * CHUNK)) * CHUNK
    n_pad = NW * p_per_w
    pad = n_pad - n

    xp = jnp.pad(x_idx, (0, pad), constant_values=GRID // 2)
    yp = jnp.pad(y_idx, (0, pad), constant_values=GRID // 2)
    zp = jnp.pad(z_idx, (0, pad), constant_values=GRID // 2)
    nxp = jnp.pad(normal_x, (0, pad), constant_values=1.0)
    nyp = jnp.pad(normal_y, (0, pad), constant_values=1.0)
    nzp = jnp.pad(normal_z, (0, pad), constant_values=1.0)

    sin_rows = subdomain_in.reshape(BATCH, NROWS, ROWW)
    sout_rows = subdomain_out.reshape(BATCH, NROWS, ROWW)

    mesh = plsc.VectorSubcoreMesh(core_axis_name="c", subcore_axis_name="s")
    fn = functools.partial(_sc_body, n, p_per_w)
    partials = pl.kernel(
        fn,
        out_type=jax.ShapeDtypeStruct((NW, L), jnp.float32),
        mesh=mesh,
        compiler_params=pltpu.CompilerParams(use_tc_tiling_on_sc=False, needs_layout_passes=False),
        scratch_types=[
            pltpu.VMEM((p_per_w,), jnp.int32),
            pltpu.VMEM((p_per_w,), jnp.int32),
            pltpu.VMEM((p_per_w,), jnp.int32),
            pltpu.VMEM((p_per_w,), jnp.float32),
            pltpu.VMEM((p_per_w,), jnp.float32),
            pltpu.VMEM((p_per_w,), jnp.float32),
            pltpu.VMEM((p_per_w,), jnp.int32),
            pltpu.VMEM((p_per_w,), jnp.int32),
            pltpu.VMEM((p_per_w,), jnp.int32),
            pltpu.VMEM((p_per_w,), jnp.int32),
            pltpu.VMEM((p_per_w,), jnp.int32),
            pltpu.VMEM((p_per_w,), jnp.int32),
            pltpu.VMEM((p_per_w,), jnp.int32),
            pltpu.VMEM((p_per_w,), jnp.int32),
            pltpu.VMEM((p_per_w,), jnp.int32),
            pltpu.VMEM((p_per_w,), jnp.int32),
            pltpu.VMEM((p_per_w,), jnp.int32),
            pltpu.VMEM((p_per_w,), jnp.int32),
            pltpu.VMEM((p_per_w,), jnp.int32),
            pltpu.VMEM((p_per_w,), jnp.int32),
            pltpu.VMEM((CHUNK, ROWW), jnp.float32),
            pltpu.VMEM((CHUNK, ROWW), jnp.float32),
            pltpu.VMEM((CHUNK, ROWW), jnp.float32),
            pltpu.VMEM((CHUNK, ROWW), jnp.float32),
            pltpu.VMEM((CHUNK, ROWW), jnp.float32),
            pltpu.VMEM((CHUNK, ROWW), jnp.float32),
            pltpu.VMEM((CHUNK, ROWW), jnp.float32),
            pltpu.VMEM((CHUNK, ROWW), jnp.float32),
            pltpu.VMEM((L,), jnp.float32),
            pltpu.SemaphoreType.DMA,
        ],
    )(sin_rows, sout_rows, xp, yp, zp, nxp, nyp, nzp)

    scale = jnp.float32(WEIGHT / (BATCH * n))
    return jnp.sum(partials) * scale
